# baseline (speedup 1.0000x reference)
import functools

import jax
import jax.numpy as jnp
from jax import lax
from jax.experimental import pallas as pl
from jax.experimental.pallas import tpu as pltpu
from jax.experimental.pallas import tpu_sc as plsc

D_MODEL = 1024
MAX_LEN = 300
SEQ = 8192

NUM_CORES = 2
NUM_SUBCORES = 16
NUM_WORKERS = NUM_CORES * NUM_SUBCORES
B_PER_W = SEQ // NUM_WORKERS
CHUNK = 32
N_CHUNKS = B_PER_W // CHUNK
LANES = 16


def _sc_body(src_hbm, sil_hbm, pe_hbm, out_hbm, sil_v, idx_v, srcbuf_v, pebuf_v,
             sem_a, sem_b):
    wid = lax.axis_index("s") * NUM_CORES + lax.axis_index("c")
    base = wid * B_PER_W

    pltpu.sync_copy(sil_hbm.at[pl.ds(base, B_PER_W)], sil_v)

    for k in range(B_PER_W // LANES):
        s = sil_v[pl.ds(k * LANES, LANES)]
        idx_v[pl.ds(k * LANES, LANES)] = jnp.where(
            s > 0, jnp.minimum(s, MAX_LEN - 1), MAX_LEN
        )

    def add_row(r, _):
        for k in range(D_MODEL // LANES):
            sl = pl.ds(k * LANES, LANES)
            srcbuf_v[r, sl] = srcbuf_v[r, sl] + pebuf_v[r, sl]
        return 0

    for c in range(N_CHUNKS):
        off = base + c * CHUNK
        cp_src = pltpu.async_copy(src_hbm.at[pl.ds(off, CHUNK)], srcbuf_v, sem_a)
        cp_pe = pltpu.async_copy(
            pe_hbm.at[idx_v.at[pl.ds(c * CHUNK, CHUNK)]], pebuf_v, sem_b
        )
        cp_src.wait()
        cp_pe.wait()
        lax.fori_loop(0, CHUNK, add_row, 0)
        pltpu.sync_copy(srcbuf_v, out_hbm.at[pl.ds(off, CHUNK)])


@functools.partial(jax.jit, static_argnames=())
def _run(src2d, sil, pe_pad):
    mesh = plsc.VectorSubcoreMesh(core_axis_name="c", subcore_axis_name="s")
    fn = pl.kernel(
        _sc_body,
        out_type=jax.ShapeDtypeStruct((SEQ, D_MODEL), jnp.float32),
        mesh=mesh,
        scratch_types=[
            pltpu.VMEM((B_PER_W,), jnp.int32),
            pltpu.VMEM((B_PER_W,), jnp.int32),
            pltpu.VMEM((CHUNK, D_MODEL), jnp.float32),
            pltpu.VMEM((CHUNK, D_MODEL), jnp.float32),
            pltpu.SemaphoreType.DMA,
            pltpu.SemaphoreType.DMA,
        ],
    )
    return fn(src2d, sil, pe_pad)


def kernel(src, silence, pe):
    src2d = src.reshape(SEQ, D_MODEL)
    sil = silence.astype(jnp.int32)
    pe_pad = jnp.concatenate(
        [pe.astype(jnp.float32), jnp.zeros((1, D_MODEL), jnp.float32)], axis=0
    )
    out = _run(src2d, sil, pe_pad)
    return out.reshape(1, SEQ, D_MODEL)


# speedup vs baseline: 1.0872x; 1.0872x over previous
import functools

import jax
import jax.numpy as jnp
from jax import lax
from jax.experimental import pallas as pl
from jax.experimental.pallas import tpu as pltpu
from jax.experimental.pallas import tpu_sc as plsc

D_MODEL = 1024
MAX_LEN = 300
SEQ = 8192

NUM_CORES = 2
NUM_SUBCORES = 16
NUM_WORKERS = NUM_CORES * NUM_SUBCORES
B_PER_W = SEQ // NUM_WORKERS
CHUNK = 16
N_CHUNKS = B_PER_W // CHUNK
LANES = 16


def _sc_body(src_hbm, sil_hbm, pe_hbm, out_hbm, sil_v, idx_v, srcbuf_v, pebuf_v,
             sem_src, sem_pe, sem_out):
    wid = lax.axis_index("s") * NUM_CORES + lax.axis_index("c")
    base = wid * B_PER_W

    pltpu.sync_copy(sil_hbm.at[pl.ds(base, B_PER_W)], sil_v)

    for k in range(B_PER_W // LANES):
        s = sil_v[pl.ds(k * LANES, LANES)]
        idx_v[pl.ds(k * LANES, LANES)] = jnp.where(
            s > 0, jnp.minimum(s, MAX_LEN - 1), MAX_LEN
        )

    def start_loads(c, b):
        off = base + c * CHUNK
        pltpu.async_copy(src_hbm.at[pl.ds(off, CHUNK)], srcbuf_v.at[b],
                         sem_src.at[b])
        pltpu.async_copy(pe_hbm.at[idx_v.at[pl.ds(c * CHUNK, CHUNK)]],
                         pebuf_v.at[b], sem_pe.at[b])

    def wait_loads(c, b):
        off = base + c * CHUNK
        pltpu.make_async_copy(src_hbm.at[pl.ds(off, CHUNK)], srcbuf_v.at[b],
                              sem_src.at[b]).wait()
        pltpu.make_async_copy(pe_hbm.at[idx_v.at[pl.ds(c * CHUNK, CHUNK)]],
                              pebuf_v.at[b], sem_pe.at[b]).wait()

    def make_add_row(b):
        def add_row(r, _):
            for k in range(D_MODEL // LANES):
                sl = pl.ds(k * LANES, LANES)
                srcbuf_v[b, r, sl] = srcbuf_v[b, r, sl] + pebuf_v[b, r, sl]
            return 0
        return add_row

    start_loads(0, 0)
    for c in range(N_CHUNKS):
        cur = c % 2
        nxt = 1 - cur
        if c + 1 < N_CHUNKS:
            if c >= 1:
                pltpu.make_async_copy(
                    srcbuf_v.at[nxt],
                    out_hbm.at[pl.ds(base + (c - 1) * CHUNK, CHUNK)],
                    sem_out.at[nxt],
                ).wait()
            start_loads(c + 1, nxt)
        wait_loads(c, cur)
        lax.fori_loop(0, CHUNK, make_add_row(cur), 0)
        pltpu.async_copy(srcbuf_v.at[cur],
                         out_hbm.at[pl.ds(base + c * CHUNK, CHUNK)],
                         sem_out.at[cur])
    for c in (N_CHUNKS - 2, N_CHUNKS - 1):
        b = c % 2
        pltpu.make_async_copy(srcbuf_v.at[b],
                              out_hbm.at[pl.ds(base + c * CHUNK, CHUNK)],
                              sem_out.at[b]).wait()


@functools.partial(jax.jit, static_argnames=())
def _run(src2d, sil, pe_pad):
    mesh = plsc.VectorSubcoreMesh(core_axis_name="c", subcore_axis_name="s")
    fn = pl.kernel(
        _sc_body,
        out_type=jax.ShapeDtypeStruct((SEQ, D_MODEL), jnp.float32),
        mesh=mesh,
        scratch_types=[
            pltpu.VMEM((B_PER_W,), jnp.int32),
            pltpu.VMEM((B_PER_W,), jnp.int32),
            pltpu.VMEM((2, CHUNK, D_MODEL), jnp.float32),
            pltpu.VMEM((2, CHUNK, D_MODEL), jnp.float32),
            pltpu.SemaphoreType.DMA((2,)),
            pltpu.SemaphoreType.DMA((2,)),
            pltpu.SemaphoreType.DMA((2,)),
        ],
    )
    return fn(src2d, sil, pe_pad)


def kernel(src, silence, pe):
    src2d = src.reshape(SEQ, D_MODEL)
    sil = silence.astype(jnp.int32)
    pe_pad = jnp.concatenate(
        [pe.astype(jnp.float32), jnp.zeros((1, D_MODEL), jnp.float32)], axis=0
    )
    out = _run(src2d, sil, pe_pad)
    return out.reshape(1, SEQ, D_MODEL)
